# Initial kernel scaffold; baseline (speedup 1.0000x reference)
import jax
import jax.numpy as jnp
from jax.experimental import pallas as pl


def kernel(pts3D, src):
    raise NotImplementedError("write your pallas kernel here")


# speedup vs baseline: 900.8322x; 900.8322x over previous
import jax
import jax.numpy as jnp
import numpy as np
from jax import lax
from jax.experimental import pallas as pl
from jax.experimental.pallas import tpu as pltpu
from jax.experimental.pallas import tpu_sc as plsc

B, N, C, S = 4, 65536, 16, 256
P = S * S
L = 16
NT = 16
NPT = N // NT
NPQ = NPT // 4
CAPT = 38912
CH = 512
PADR = 512
NPIX = P + PADR
ROWS_T = NPIX // NT
CAPOF = 1024
OUTW = NPIX * C
RING = 2048

RADIUS_NDC = float(1.5) / float(S) * 2.0
R2 = np.float32(RADIUS_NDC ** 2)
SIGN = np.int32(-2147483648)


def _iota():
    return lax.broadcasted_iota(jnp.int32, (L,), 0)


def _ds16(x):
    return pl.ds(pl.multiple_of(x, 16), L)


def _take16(arr, idx):
    dn = lax.GatherDimensionNumbers(offset_dims=(), collapsed_slice_dims=(0,),
                                    start_index_map=(0,))
    return lax.gather(arr, idx[:, None], dimension_numbers=dn,
                      slice_sizes=(1,),
                      mode=lax.GatherScatterMode.PROMISE_IN_BOUNDS)


def _floor_i32(v):
    t = v.astype(jnp.int32)
    return jnp.where(v < t.astype(jnp.float32), t - 1, t)


def _sqrt(v):
    i = plsc.bitcast(v, jnp.int32)
    y = plsc.bitcast(jnp.int32(0x1FBD1DF5) + (i >> 1), jnp.float32)
    for _ in range(3):
        y = 0.5 * (y + v / y)
    return y


def _body(xs, ys, zs, src, outT, pk_hbm,
          xv, yv, zv, featT, stg_v, pkc_v, wb_v, zc4_v,
          idx_st, cnt_ch, ones_st, wv_st, pid16_st, lptl_st, col_st, idxc_st,
          idxg_v, ofp_v, ofz_v, ofs_v, plp_v, plz_v, pls_v, plk_v, st16,
          count_sh, out_sh, pool_p, pool_z, pool_s, pool_k, cnt_sm):
    c = lax.axis_index("c")
    s = lax.axis_index("s")
    gbase = (c * NT + s) * CAPT

    def z1_step(t, carry9):
        zc4_v[pl.ds(t * L, L)] = jnp.zeros((L,), jnp.float32)
        return carry9

    lax.fori_loop(0, ROWS_T // L, z1_step, jnp.int32(0))

    def z2_step(t, carry9):
        wb_v[pl.ds(t * L, L)] = jnp.zeros((L,), jnp.float32)
        return carry9

    lax.fori_loop(0, 4096 // L, z2_step, jnp.int32(0))
    for r in range(4):
        def o1_step(t, carry9):
            ones_st[r, pl.ds(t * L, L)] = jnp.full((L,), 1.0, jnp.float32)
            return carry9
        lax.fori_loop(0, 8, o1_step, jnp.int32(0))

    def one_batch(ib, carry):
        b = 2 * c + ib
        base_pt = b * N + s * NPT

        pltpu.sync_copy(zc4_v, count_sh.at[pl.ds(s * ROWS_T, ROWS_T)])
        def zo_step(k, carry9):
            pltpu.sync_copy(wb_v, out_sh.at[pl.ds(
                pl.multiple_of((s * 16 + k) * 4096, 4096), 4096)])
            return carry9

        lax.fori_loop(0, 16, zo_step, jnp.int32(0))
        pltpu.sync_copy(zc4_v.at[pl.ds(0, 512)],
                        out_sh.at[pl.ds(NT * 16 * 4096 + s * 512, 512)])
        cnt_sm[0] = 0
        plsc.subcore_barrier()

        pltpu.sync_copy(xs.at[pl.ds(base_pt, NPT)], xv)
        pltpu.sync_copy(ys.at[pl.ds(base_pt, NPT)], yv)
        pltpu.sync_copy(zs.at[pl.ds(base_pt, NPT)], zv)

        def flush(off, fl):
            need = (off - fl) >= CH

            @pl.when(need)
            def _():
                pltpu.sync_copy(
                    stg_v.at[pl.ds(pl.multiple_of(fl & (RING - 1), CH), CH)],
                    pk_hbm.at[pl.ds(pl.multiple_of(gbase + fl, CH), CH)])
            return jnp.where(need, fl + CH, fl)

        def gen_step(i, carry2):
            off, fl = carry2
            x = -xv[pl.ds(i * L, L)]
            y = -yv[pl.ds(i * L, L)]
            z = zv[pl.ds(i * L, L)]
            jf = (1.0 - x) * 128.0 - 0.5
            if_ = (1.0 - y) * 128.0 - 0.5
            cj = _floor_i32(jnp.clip(jf, -100.0, 400.0) + 0.5)
            ci = _floor_i32(jnp.clip(if_, -100.0, 400.0) + 0.5)
            zpos = z > 0.0
            lpt = i * L + _iota()
            def off_step(o, off2):
                di = o // 3 - 1
                dj = o - 3 * (o // 3) - 1
                pi = ci + di
                pj = cj + dj
                cx = 1.0 - (pj.astype(jnp.float32) + 0.5) * np.float32(2.0 / S)
                cy = 1.0 - (pi.astype(jnp.float32) + 0.5) * np.float32(2.0 / S)
                dx = x - cx
                dy = y - cy
                d2 = dx * dx + dy * dy
                vd = ((pi >= 0) & (pi < S) & (pj >= 0) & (pj < S)
                      & (d2 < R2) & zpos)
                vdi = vd.astype(jnp.int32)
                pos = (off2 + jnp.cumsum(vdi) - 1) & (RING - 1)
                pack = (pi * S + pj) * 4096 + lpt
                plsc.store_scatter(stg_v, [pos], pack, mask=vd)
                return off2 + jnp.sum(vdi)

            off = lax.fori_loop(0, 9, off_step, off)
            fl = flush(off, fl)
            return off, fl

        bounds = []
        off = jnp.int32(0)
        fl = jnp.int32(0)
        for q in range(4):
            off, fl = lax.fori_loop(q * (NPQ // L), (q + 1) * (NPQ // L),
                                    gen_step, (off, fl))
            npad = ((off + CH - 1) // CH) * CH
            for j in range(CH // L):
                posp = off + j * L + _iota()
                mp = posp < npad
                padpid = P + (posp & (PADR - 1))
                plsc.store_scatter(stg_v, [posp & (RING - 1)],
                                   padpid * 4096, mask=mp)
            off = npad

            def fin_step(u, fl2):
                return flush(off, fl2)

            fl = lax.fori_loop(0, (off - fl) // CH, fin_step, fl)
            bounds.append(npad // CH)
        nch = bounds[3]

        def load_chunk(j):
            pltpu.sync_copy(
                pk_hbm.at[pl.ds(pl.multiple_of(gbase + j * CH, CH), CH)],
                pkc_v)

        def cnt_step(j, carry2):
            load_chunk(j)
            for r in range(4):
                def ib_step(t, carry9):
                    pk = pkc_v[pl.ds(r * 128 + t * L, L)]
                    idx_st[r, pl.ds(t * L, L)] = (pk & 0x7FFFFFFF) >> 12
                    return carry9
                lax.fori_loop(0, 8, ib_step, jnp.int32(0))
            for r in range(4):
                pltpu.sync_copy(ones_st.at[r], count_sh.at[idx_st.at[r]],
                                add=True)
            return carry2

        lax.fori_loop(0, nch, cnt_step, jnp.int32(0))
        plsc.subcore_barrier()

        def flag_step(j, m):
            load_chunk(j)
            for r in range(4):
                def ib_step(t, carry9):
                    pk = pkc_v[pl.ds(r * 128 + t * L, L)]
                    idx_st[r, pl.ds(t * L, L)] = (pk & 0x7FFFFFFF) >> 12
                    return carry9
                lax.fori_loop(0, 8, ib_step, jnp.int32(0))
            for r in range(4):
                pltpu.sync_copy(count_sh.at[idx_st.at[r]], cnt_ch.at[r])
            for r in range(4):
                def flag_inner(t, m2):
                    basek = r * 128 + t * L
                    pk = pkc_v[pl.ds(basek, L)]
                    pidv = pk >> 12
                    lptv = pk & 4095
                    cn = cnt_ch[r, pl.ds(t * L, L)]
                    fl2 = (cn > 8.5) & (pidv < P)
                    pos = m2 + jnp.cumsum(fl2.astype(jnp.int32)) - 1
                    fl2 = fl2 & (pos < CAPOF)
                    zk = plsc.bitcast(plsc.load_gather(zv, [lptv]), jnp.int32)
                    slot = gbase + j * CH + basek + _iota()
                    plsc.store_scatter(ofp_v, [pos], pidv, mask=fl2)
                    plsc.store_scatter(ofz_v, [pos], zk, mask=fl2)
                    plsc.store_scatter(ofs_v, [pos], slot, mask=fl2)
                    return m2 + jnp.sum(fl2.astype(jnp.int32))
                m = lax.fori_loop(0, 8, flag_inner, m)
            return m

        m = lax.fori_loop(0, nch, flag_step, jnp.int32(0))
        mr = (m + 15) & ~jnp.int32(15)
        posp = m + _iota()
        mp = (posp < mr) & (posp < CAPOF)
        plsc.store_scatter(ofp_v, [posp],
                           jnp.full((L,), P + 4095, jnp.int32), mask=mp)
        plsc.store_scatter(ofz_v, [posp], jnp.zeros((L,), jnp.int32), mask=mp)
        plsc.store_scatter(ofs_v, [posp], jnp.zeros((L,), jnp.int32), mask=mp)
        base = plsc.fetch_and_add(cnt_sm.at[0], mr, subcore_id=0)

        def pub_step(u, carry2):
            src_sl = _ds16(u * L)
            dst_sl = _ds16(base + u * L)
            pltpu.sync_copy(ofp_v.at[src_sl], pool_p.at[dst_sl])
            pltpu.sync_copy(ofz_v.at[src_sl], pool_z.at[dst_sl])
            pltpu.sync_copy(ofs_v.at[src_sl], pool_s.at[dst_sl])
            return carry2

        npub = jnp.where(base + mr > CAPOF, jnp.maximum(CAPOF - base, 0), mr)
        lax.fori_loop(0, npub // L, pub_step, jnp.int32(0))
        plsc.subcore_barrier()

        n_of = jnp.minimum(
            plsc.fetch_and_add(cnt_sm.at[0], 0, subcore_id=0), CAPOF)
        nv = (n_of + 15) >> 4

        def pool_cp(u, carry2):
            sl = _ds16(u * L)
            pltpu.sync_copy(pool_p.at[sl], plp_v.at[sl])
            pltpu.sync_copy(pool_z.at[sl], plz_v.at[sl])
            pltpu.sync_copy(pool_s.at[sl], pls_v.at[sl])
            return carry2

        lax.fori_loop(0, nv, pool_cp, jnp.int32(0))

        def rank_step(v, carry2):
            @pl.when((v & (NT - 1)) == s)
            def _():
                rp = plp_v[pl.ds(v * L, L)]
                rz = plz_v[pl.ds(v * L, L)]
                rs = pls_v[pl.ds(v * L, L)]

                def scan_w(w, rank):
                    qp = plp_v[pl.ds(w * L, L)]
                    qz = plz_v[pl.ds(w * L, L)]
                    qs = pls_v[pl.ds(w * L, L)]
                    def perm_step(k, rank2):
                        perm = (_iota() + k) & (L - 1)
                        qpk = _take16(qp, perm)
                        qzk = _take16(qz, perm)
                        qsk = _take16(qs, perm)
                        less = (qpk == rp) & ((qzk < rz)
                                              | ((qzk == rz) & (qsk < rs)))
                        return rank2 + less.astype(jnp.int32)

                    return lax.fori_loop(0, L, perm_step, rank)

                rank = lax.fori_loop(0, nv, scan_w, jnp.zeros((L,), jnp.int32))
                st16[pl.ds(0, L)] = (rank >= 8).astype(jnp.int32)
                pltpu.sync_copy(st16, pool_k.at[_ds16(v * L)])
            return carry2

        lax.fori_loop(0, nv, rank_step, jnp.int32(0))
        plsc.subcore_barrier()

        def drop_step(v, carry2):
            sl = _ds16(v * L)
            pltpu.sync_copy(pool_k.at[sl], plk_v.at[sl])
            sp = pls_v[sl]
            kp = plk_v[sl]
            pp = plp_v[sl]
            mine = ((sp >= gbase) & (sp < gbase + CAPT)
                    & (pp < P) & (kp > 0))
            idxg_v[pl.ds(0, L)] = jnp.where(mine, sp, gbase + CH - 1)
            pltpu.sync_copy(pk_hbm.at[idxg_v], st16)
            old = st16[pl.ds(0, L)]
            st16[pl.ds(0, L)] = jnp.where(mine, old | SIGN, old)
            pltpu.sync_copy(st16, pk_hbm.at[idxg_v])
            return carry2

        lax.fori_loop(0, nv, drop_step, jnp.int32(0))

        for q in range(4):
            for ch in range(C):
                pltpu.sync_copy(
                    src.at[b, ch, pl.ds(s * NPT + q * NPQ, NPQ)],
                    featT.at[ch])

            def acc_step(j, carry2):
                load_chunk(j)
                for r in range(4):
                    def w_step(t, carry4):
                        pk = pkc_v[pl.ds(r * 128 + t * L, L)]
                        dropped = pk < 0
                        pku = pk & 0x7FFFFFFF
                        pidv = pku >> 12
                        lptv = pku & 4095
                        x = -plsc.load_gather(xv, [lptv])
                        y = -plsc.load_gather(yv, [lptv])
                        pj = pidv & 255
                        pi = (pidv >> 8) & 511
                        cx = 1.0 - (pj.astype(jnp.float32) + 0.5) * np.float32(2.0 / S)
                        cy = 1.0 - (pi.astype(jnp.float32) + 0.5) * np.float32(2.0 / S)
                        dx = x - cx
                        dy = y - cy
                        d2 = dx * dx + dy * dy
                        w = 1.0 - _sqrt(jnp.clip(d2 / R2, 0.001, 1.0))
                        wzero = dropped | (pidv >= P)
                        w = jnp.where(wzero, 0.0, w)
                        sl = pl.ds(t * L, L)
                        wv_st[r, sl] = w
                        pid16_st[r, sl] = pidv * 16
                        lptl_st[r, sl] = jnp.where(wzero, 0, lptv - q * NPQ)
                        return carry4
                    lax.fori_loop(0, 8, w_step, jnp.int32(0))
                def ch_step(ch, carry3):
                    chv = jnp.full((L,), 0, jnp.int32) + ch
                    for r in range(4):
                        def col_step(t, carry4):
                            sl = pl.ds(t * L, L)
                            lptl = lptl_st[r, sl]
                            val = plsc.load_gather(featT, [chv, lptl])
                            col_st[r, sl] = val * wv_st[r, sl]
                            idxc_st[r, sl] = pid16_st[r, sl] + ch
                            return carry4
                        lax.fori_loop(0, 8, col_step, jnp.int32(0))
                    for r2 in range(4):
                        pltpu.sync_copy(col_st.at[r2],
                                        out_sh.at[idxc_st.at[r2]], add=True)
                    return carry3

                lax.fori_loop(0, C, ch_step, jnp.int32(0))
                return carry2

            lo = jnp.int32(0) if q == 0 else bounds[q - 1]
            lax.fori_loop(lo, bounds[q], acc_step, jnp.int32(0))
        plsc.subcore_barrier()

        def wo_step(k, carry9):
            sl = pl.ds(pl.multiple_of((s * 16 + k) * 4096, 4096), 4096)
            pltpu.sync_copy(out_sh.at[sl], wb_v)
            pltpu.sync_copy(wb_v, outT.at[pl.ds(
                pl.multiple_of(b * P * C + (s * 16 + k) * 4096, 4096), 4096)])
            return carry9

        lax.fori_loop(0, 16, wo_step, jnp.int32(0))
        def z3_step(t, carry9):
            wb_v[pl.ds(t * L, L)] = jnp.zeros((L,), jnp.float32)
            return carry9

        lax.fori_loop(0, 4096 // L, z3_step, jnp.int32(0))
        plsc.subcore_barrier()
        return carry

    lax.fori_loop(0, 2, one_batch, jnp.int32(0))


@jax.jit
def kernel(pts3D, src):
    xs = pts3D[..., 0].reshape(B * N)
    ys = pts3D[..., 1].reshape(B * N)
    zs = pts3D[..., 2].reshape(B * N)
    mesh = plsc.VectorSubcoreMesh(core_axis_name="c", subcore_axis_name="s")
    f = pl.kernel(
        _body,
        out_type=(jax.ShapeDtypeStruct((B * P * C,), jnp.float32),
                  jax.ShapeDtypeStruct((2 * NT * CAPT,), jnp.int32)),
        mesh=mesh,
        compiler_params=pltpu.CompilerParams(needs_layout_passes=False),
        scratch_types=[
            pltpu.VMEM((NPT,), jnp.float32),
            pltpu.VMEM((NPT,), jnp.float32),
            pltpu.VMEM((NPT,), jnp.float32),
            pltpu.VMEM((C, NPQ), jnp.float32),
            pltpu.VMEM((RING,), jnp.int32),
            pltpu.VMEM((CH,), jnp.int32),
            pltpu.VMEM((4096,), jnp.float32),
            pltpu.VMEM((ROWS_T,), jnp.float32),
            pltpu.VMEM((4, 128), jnp.int32),
            pltpu.VMEM((4, 128), jnp.float32),
            pltpu.VMEM((4, 128), jnp.float32),
            pltpu.VMEM((4, 128), jnp.float32),
            pltpu.VMEM((4, 128), jnp.int32),
            pltpu.VMEM((4, 128), jnp.int32),
            pltpu.VMEM((4, 128), jnp.float32),
            pltpu.VMEM((4, 128), jnp.int32),
            pltpu.VMEM((L,), jnp.int32),
            pltpu.VMEM((CAPOF,), jnp.int32),
            pltpu.VMEM((CAPOF,), jnp.int32),
            pltpu.VMEM((CAPOF,), jnp.int32),
            pltpu.VMEM((CAPOF,), jnp.int32),
            pltpu.VMEM((CAPOF,), jnp.int32),
            pltpu.VMEM((CAPOF,), jnp.int32),
            pltpu.VMEM((CAPOF,), jnp.int32),
            pltpu.VMEM((L,), jnp.int32),
            pltpu.VMEM_SHARED((NPIX,), jnp.float32),
            pltpu.VMEM_SHARED((OUTW,), jnp.float32),
            pltpu.VMEM_SHARED((CAPOF,), jnp.int32),
            pltpu.VMEM_SHARED((CAPOF,), jnp.int32),
            pltpu.VMEM_SHARED((CAPOF,), jnp.int32),
            pltpu.VMEM_SHARED((CAPOF,), jnp.int32),
            pltpu.SMEM((1,), jnp.int32),
        ],
    )
    outT, _ = f(xs, ys, zs, src)
    return outT.reshape(B, S, S, C).transpose(0, 3, 1, 2)
